# baseline (speedup 1.0000x reference)
import jax
import jax.numpy as jnp
from jax.experimental import pallas as pl
from jax.experimental.pallas import tpu as pltpu

N = 10000
D_IN = 128
D_HID = 16
D_OUT = 7
BM = 400
GRID = N // BM


def _dot(a, b):
    return jax.lax.dot_general(a, b, (((1,), (0,)), ((), ())),
                               preferred_element_type=jnp.float32)


def _gcn_kernel(a_ref, x_ref, w1_ref, b1_ref, w2_ref, b2_ref, out_ref,
                s1_ref, s2_ref):
    p = pl.program_id(0)
    i = pl.program_id(1)

    @pl.when((p == 0) & (i == 0))
    def _():
        s1_ref[...] = _dot(x_ref[...], w1_ref[...]).astype(jnp.bfloat16)

    a16 = a_ref[...].astype(jnp.bfloat16)

    @pl.when(p == 0)
    def _():
        h = jnp.maximum(_dot(a16, s1_ref[...]) + b1_ref[...], 0.0)
        s2_ref[pl.ds(i * BM, BM), :] = _dot(
            h, w2_ref[...]).astype(jnp.bfloat16)

    @pl.when(p == 1)
    def _():
        out_ref[...] = _dot(a16, s2_ref[...]) + b2_ref[...]


def kernel(adjacency, feature, W1, b1, W2, b2):
    return pl.pallas_call(
        _gcn_kernel,
        grid=(2, GRID),
        in_specs=[
            pl.BlockSpec((BM, N), lambda p, i: (i, 0)),
            pl.BlockSpec((N, D_IN), lambda p, i: (0, 0)),
            pl.BlockSpec((D_IN, D_HID), lambda p, i: (0, 0)),
            pl.BlockSpec((1, D_HID), lambda p, i: (0, 0)),
            pl.BlockSpec((D_HID, D_OUT), lambda p, i: (0, 0)),
            pl.BlockSpec((1, D_OUT), lambda p, i: (0, 0)),
        ],
        out_specs=pl.BlockSpec((BM, D_OUT), lambda p, i: (i, 0)),
        out_shape=jax.ShapeDtypeStruct((N, D_OUT), jnp.float32),
        scratch_shapes=[
            pltpu.VMEM((N, D_HID), jnp.bfloat16),
            pltpu.VMEM((N, D_OUT), jnp.bfloat16),
        ],
    )(adjacency, feature, W1, b1.reshape(1, D_HID), W2,
      b2.reshape(1, D_OUT))


# speedup vs baseline: 1.0322x; 1.0322x over previous
import jax
import jax.numpy as jnp
from jax.experimental import pallas as pl
from jax.experimental.pallas import tpu as pltpu

N = 10000
D_IN = 128
D_HID = 16
D_OUT = 7
BM = 400
GRID = N // BM


def _dot(a, b, precision=None):
    return jax.lax.dot_general(a, b, (((1,), (0,)), ((), ())),
                               precision=precision,
                               preferred_element_type=jnp.float32)


def _gcn_kernel(a_ref, x_ref, w1_ref, b1_ref, w2_ref, b2_ref, out_ref,
                s1_ref, s2_ref):
    p = pl.program_id(0)
    i = pl.program_id(1)

    @pl.when((p == 0) & (i == 0))
    def _():
        s1_ref[...] = _dot(x_ref[...], w1_ref[...]).astype(jnp.bfloat16)

    @pl.when(p == 0)
    def _():
        a16 = a_ref[...].astype(jnp.bfloat16)
        h = jnp.maximum(_dot(a16, s1_ref[...]) + b1_ref[...], 0.0)
        s2_ref[pl.ds(i * BM, BM), :] = _dot(
            h, w2_ref[...]).astype(jnp.bfloat16)

    @pl.when(p == 1)
    def _():
        a16 = a_ref[...].astype(jnp.bfloat16)
        out_ref[...] = _dot(a16, s2_ref[...]) + b2_ref[...]


def kernel(adjacency, feature, W1, b1, W2, b2):
    return pl.pallas_call(
        _gcn_kernel,
        grid=(2, GRID),
        in_specs=[
            pl.BlockSpec((BM, N), lambda p, i: (i, 0)),
            pl.BlockSpec((N, D_IN), lambda p, i: (0, 0)),
            pl.BlockSpec((D_IN, D_HID), lambda p, i: (0, 0)),
            pl.BlockSpec((1, D_HID), lambda p, i: (0, 0)),
            pl.BlockSpec((D_HID, D_OUT), lambda p, i: (0, 0)),
            pl.BlockSpec((1, D_OUT), lambda p, i: (0, 0)),
        ],
        out_specs=pl.BlockSpec((BM, D_OUT), lambda p, i: (i, 0)),
        out_shape=jax.ShapeDtypeStruct((N, D_OUT), jnp.float32),
        scratch_shapes=[
            pltpu.VMEM((N, D_HID), jnp.bfloat16),
            pltpu.VMEM((N, D_OUT), jnp.bfloat16),
        ],
    )(adjacency, feature, W1, b1.reshape(1, D_HID), W2,
      b2.reshape(1, D_OUT))
